# baseline (speedup 1.0000x reference)
import functools
import math

import jax
import jax.numpy as jnp
from jax import lax
from jax.experimental import pallas as pl
from jax.experimental.pallas import tpu as pltpu
from jax.experimental.pallas import tpu_sc as plsc

_B, _S, _N, _F = 1, 8, 10000, 128
_HID, _TDIM, _FH = 128, 64, 4
_E = 160000

_BN = 512
_NPAD = 10240
_NB = _NPAD // _BN
_NSC = 16
_CHUNK = 128
_NBUF = 6
_NDEF = 3
_NCH = 84
_EPT = _NCH * _CHUNK
_EPAD = _NSC * _EPT
_RPT = _NPAD // _NSC


def _gelu(v):
    return 0.5 * v * (1.0 + lax.erf(v * (1.0 / math.sqrt(2.0))))


def _transform_body(split_in, h_ref, w_ref, b_ref, tt_ref, s_ref):
    if split_in:
        g = _gelu(jnp.concatenate([h_ref[0, 0], h_ref[0, 1]], axis=1))
    else:
        g = h_ref[0]
    r0 = jnp.dot(g, w_ref[0], preferred_element_type=jnp.float32) + b_ref[0][None, :]
    r1 = jnp.dot(g, w_ref[1], preferred_element_type=jnp.float32) + b_ref[1][None, :]
    r2 = jnp.dot(g, w_ref[2], preferred_element_type=jnp.float32) + b_ref[2][None, :]
    tt_ref[0, 0] = r0
    tt_ref[0, 1] = r1
    s_ref[0, 0] = r2[:, :_HID // 2]
    s_ref[0, 1] = r2[:, _HID // 2:]


def _transform(h, w_all, b_all, split_in):
    if split_in:
        h_spec = pl.BlockSpec((1, 2, _BN, _HID // 2), lambda t, n: (t, 0, n, 0))
    else:
        h_spec = pl.BlockSpec((1, _BN, _HID), lambda t, n: (t, n, 0))
    return pl.pallas_call(
        functools.partial(_transform_body, split_in),
        grid=(_S, _NB),
        in_specs=[
            h_spec,
            pl.BlockSpec((3, _HID, _HID), lambda t, n: (0, 0, 0)),
            pl.BlockSpec((3, _HID), lambda t, n: (0, 0)),
        ],
        out_specs=[
            pl.BlockSpec((1, 2, _BN, _HID), lambda t, n: (t, 0, n, 0)),
            pl.BlockSpec((1, 2, _BN, _HID // 2), lambda t, n: (t, 0, n, 0)),
        ],
        out_shape=[
            jax.ShapeDtypeStruct((_S, 2, _NPAD, _HID), jnp.float32),
            jax.ShapeDtypeStruct((_S, 2, _NPAD, _HID // 2), jnp.float32),
        ],
    )(h, w_all, b_all)


_HH = _HID // 2


def _sc_conv(tt_half, s_all, gidx2_all, dst3, w3):
    mesh = plsc.VectorSubcoreMesh(core_axis_name="c", subcore_axis_name="s")

    @functools.partial(
        pl.kernel,
        out_type=jax.ShapeDtypeStruct((_S, 2, _NPAD, _HH), jnp.float32),
        mesh=mesh,
        compiler_params=pltpu.CompilerParams(use_tc_tiling_on_sc=False),
        scratch_types=[
            pltpu.VMEM((_NCH, _CHUNK), jnp.int32),
            pltpu.VMEM((_NCH, _CHUNK), jnp.int32),
            pltpu.VMEM((_NCH, _CHUNK), jnp.float32),
            pltpu.VMEM((_CHUNK, _HH), jnp.float32),
            pltpu.VMEM((_CHUNK, _HH), jnp.float32),
            pltpu.VMEM((_CHUNK, _HH), jnp.float32),
            pltpu.VMEM((_CHUNK, _HH), jnp.float32),
            pltpu.VMEM((_CHUNK, _HH), jnp.float32),
            pltpu.VMEM((_CHUNK, _HH), jnp.float32),
            pltpu.VMEM_SHARED((_NPAD, _HH), jnp.float32),
            pltpu.SemaphoreType.DMA,
            pltpu.SemaphoreType.DMA,
            pltpu.SemaphoreType.DMA,
            pltpu.SemaphoreType.DMA,
            pltpu.SemaphoreType.DMA,
            pltpu.SemaphoreType.DMA,
            pltpu.SemaphoreType.DMA,
            pltpu.SemaphoreType.DMA,
            pltpu.SemaphoreType.DMA,
            pltpu.SemaphoreType.DMA,
            pltpu.SemaphoreType.DMA,
            pltpu.SemaphoreType.DMA,
        ],
    )
    def conv(tt_hbm, s_hbm, gidx_hbm, dst_hbm, w_hbm, agg_hbm,
             gi_v, di_v, w_v, r0, r1, r2, r3, r4, r5, acc,
             gs0, gs1, gs2, gs3, gs4, gs5,
             ss0, ss1, ss2, ss3, ss4, ss5):
        cid = lax.axis_index("c")
        sid = lax.axis_index("s")
        row0 = sid * _RPT
        rows = [r0, r1, r2, r3, r4, r5]
        gsem = [gs0, gs1, gs2, gs3, gs4, gs5]
        ssem = [ss0, ss1, ss2, ss3, ss4, ss5]

        pltpu.sync_copy(dst_hbm.at[sid], di_v)
        pltpu.sync_copy(w_hbm.at[sid], w_v)

        def scale(buf, i):
            def gbody(g, carry):
                w16 = w_v[i, pl.ds(g * 16, 16)]
                for j in range(16):
                    wspl = lax.gather(
                        w16, jnp.full((16, 1), j, jnp.int32),
                        lax.GatherDimensionNumbers(
                            offset_dims=(), collapsed_slice_dims=(0,),
                            start_index_map=(0,)),
                        (1,),
                        mode=lax.GatherScatterMode.PROMISE_IN_BOUNDS)
                    r = g * 16 + j
                    for fc in range(_HH // 16):
                        fsl = pl.ds(fc * 16, 16)
                        buf[r, fsl] = buf[r, fsl] * wspl
                return carry

            lax.fori_loop(0, _CHUNK // 16, gbody, 0)

        def t_body(ti, carry):
            t = cid * (_S // 2) + ti
            for h in range(2):
                pltpu.sync_copy(s_hbm.at[t, h, pl.ds(row0, _RPT)],
                                acc.at[pl.ds(row0, _RPT)])
                pltpu.sync_copy(gidx_hbm.at[t, h, sid], gi_v)
                plsc.subcore_barrier()

                for b in range(_NBUF):
                    pltpu.async_copy(tt_hbm.at[gi_v.at[b]], rows[b], gsem[b])

                def grp(i0, carry2):
                    for b in range(_NBUF):
                        i = i0 * _NBUF + b
                        bd = (b + _NDEF) % _NBUF

                        @pl.when(jnp.logical_and(i >= _NDEF,
                                                 i + _NDEF < _NCH))
                        def _():
                            pltpu.make_async_copy(
                                rows[bd], acc.at[di_v.at[i - _NDEF]],
                                ssem[bd]).wait()
                            pltpu.async_copy(
                                tt_hbm.at[gi_v.at[i + _NDEF]], rows[bd],
                                gsem[bd])

                        pltpu.make_async_copy(
                            tt_hbm.at[gi_v.at[i]], rows[b], gsem[b]).wait()
                        scale(rows[b], i)
                        pltpu.async_copy(rows[b], acc.at[di_v.at[i]],
                                         ssem[b], add=True)
                    return carry2

                lax.fori_loop(0, _NCH // _NBUF, grp, 0)
                for b in range(_NBUF):
                    pltpu.make_async_copy(
                        rows[b], acc.at[di_v.at[_NCH - _NBUF + b]],
                        ssem[b]).wait()
                plsc.subcore_barrier()
                pltpu.sync_copy(acc.at[pl.ds(row0, _RPT)],
                                agg_hbm.at[t, h, pl.ds(row0, _RPT)])
                plsc.subcore_barrier()
            return carry

        lax.fori_loop(0, _S // 2, t_body, 0)

    return conv(tt_half, s_all, gidx2_all, dst3, w3)


def _xw_body(x_ref, w_ref, b_ref, o_ref):
    o_ref[...] = (jnp.dot(x_ref[...], w_ref[...],
                          preferred_element_type=jnp.float32)
                  + b_ref[0][None, :])


def _xw(xlast, w2b, b2):
    return pl.pallas_call(
        _xw_body,
        grid=(_NB,),
        in_specs=[
            pl.BlockSpec((_BN, _F), lambda n: (n, 0)),
            pl.BlockSpec((_F, _HID), lambda n: (0, 0)),
            pl.BlockSpec((1, _HID), lambda n: (0, 0)),
        ],
        out_specs=pl.BlockSpec((_BN, _HID), lambda n: (n, 0)),
        out_shape=jax.ShapeDtypeStruct((_NPAD, _HID), jnp.float32),
    )(xlast, w2b, b2)


def _final_body(agg_ref, xw_ref, wip_ref, bip_ref, wihf_ref, whhf_ref, bf_ref,
                wihb_ref, whhb_ref, bb_ref, ng_ref, nbb_ref, w1_ref, b1_ref,
                w2a_ref, n1g_ref, n1b_ref, w3_ref, b3_ref, out_ref):
    zs = []
    for t in range(_S):
        g = _gelu(jnp.concatenate([agg_ref[t, 0], agg_ref[t, 1]], axis=1))
        zs.append(jnp.dot(g, wip_ref[...], preferred_element_type=jnp.float32)
                  + bip_ref[0][None, :])

    def lstm(zlist, wih, whh, bsum):
        h = jnp.zeros((_BN, _TDIM), jnp.float32)
        c = jnp.zeros((_BN, _TDIM), jnp.float32)
        outs = []
        for z in zlist:
            gates = (jnp.dot(z, wih, preferred_element_type=jnp.float32)
                     + jnp.dot(h, whh, preferred_element_type=jnp.float32)
                     + bsum[0][None, :])
            ii = gates[:, :_TDIM]
            ff = gates[:, _TDIM:2 * _TDIM]
            gg = gates[:, 2 * _TDIM:3 * _TDIM]
            oo = gates[:, 3 * _TDIM:]
            c = jax.nn.sigmoid(ff) * c + jax.nn.sigmoid(ii) * jnp.tanh(gg)
            h = jax.nn.sigmoid(oo) * jnp.tanh(c)
            outs.append(h)
        return outs

    fo = lstm(zs, wihf_ref[...], whhf_ref[...], bf_ref)
    bo_rev = lstm(zs[::-1], wihb_ref[...], whhb_ref[...], bb_ref)

    def ln(v, g, b):
        m = jnp.mean(v, axis=-1, keepdims=True)
        d = v - m
        var = jnp.mean(d * d, axis=-1, keepdims=True)
        return d * lax.rsqrt(var + 1e-5) * g[0][None, :] + b[0][None, :]

    rows = []
    for fh in range(_FH):
        t = (_S - _FH) + fh
        ocat = jnp.concatenate([fo[t], bo_rev[(_S - 1) - t]], axis=1)
        oln = ln(ocat, ng_ref, nbb_ref)
        u = _gelu(jnp.dot(oln, w1_ref[...], preferred_element_type=jnp.float32)
                  + b1_ref[0][None, :])
        v = (jnp.dot(u, w2a_ref[...], preferred_element_type=jnp.float32)
             + xw_ref[:, fh, :])
        y = ln(_gelu(v), n1g_ref, n1b_ref)
        rows.append(jnp.sum(y * w3_ref[0][None, :], axis=1) + b3_ref[0, 0])
    out_ref[...] = jnp.stack(rows, axis=0)


def _final(agg, xwrep, *smalls):
    small_specs = []
    for a in smalls:
        small_specs.append(
            pl.BlockSpec(a.shape, lambda n, r=len(a.shape): (0,) * r))
    return pl.pallas_call(
        _final_body,
        grid=(_NB,),
        in_specs=[
            pl.BlockSpec((_S, 2, _BN, _HID // 2), lambda n: (0, 0, n, 0)),
            pl.BlockSpec((_BN, _FH, _HID), lambda n: (n, 0, 0)),
        ] + small_specs,
        out_specs=pl.BlockSpec((_FH, _BN), lambda n: (0, n)),
        out_shape=jax.ShapeDtypeStruct((_FH, _NPAD), jnp.float32),
    )(agg, xwrep, *smalls)


def kernel(x, edge_index, edge_types, edge_weights, params):
    x = x.astype(jnp.float32)
    xp = jnp.pad(x[0], ((0, 0), (0, _NPAD - _N), (0, 0)))

    src = edge_index[0].astype(jnp.int32)
    dst = edge_index[1].astype(jnp.int32)
    ty = edge_types.astype(jnp.int32)
    pad_e = _EPAD - _E
    bidx = jnp.pad(ty * _NPAD + src, (0, pad_e))
    toffs = jnp.arange(_S, dtype=jnp.int32) * (4 * _NPAD)
    hoffs = jnp.arange(2, dtype=jnp.int32)
    gidx2_all = (2 * bidx[None, None, :] + toffs[:, None, None]
                 + hoffs[None, :, None]).reshape(_S, 2, _NSC, _NCH, _CHUNK)
    dst3 = jnp.pad(dst, (0, pad_e)).reshape(_NSC, _NCH, _CHUNK)
    w3 = jnp.pad(edge_weights.astype(jnp.float32),
                 (0, pad_e)).reshape(_NSC, _NCH, _CHUNK)

    h_in = xp
    agg = None
    for li, p in enumerate(params["gnn"]):
        att = jax.nn.softmax(p["att"])
        w_all = jnp.concatenate([p["We"], p["Ws"][None]], axis=0) * att[:, None, None]
        b_all = jnp.concatenate([p["be"], p["bs"][None]], axis=0) * att[:, None]
        tt, s_all = _transform(h_in, w_all, b_all, split_in=(li > 0))
        agg = _sc_conv(tt.reshape(_S * 2 * _NPAD * 2, _HID // 2), s_all,
                       gidx2_all, dst3, w3)
        h_in = agg

    tp = params["temporal"]
    xw = _xw(xp[_S - 1], params["W2"][_HID:], params["b2"].reshape(1, _HID))
    xwv = xw[:_N]
    xwrep = jnp.concatenate(
        [xwv, xwv, xwv, xwv, xwv[: _FH * (_NPAD - _N)]], axis=0
    ).reshape(_NPAD, _FH, _HID)

    out = _final(
        agg, xwrep,
        tp["Wip"], tp["bip"].reshape(1, _TDIM),
        tp["Wih_f"].T, tp["Whh_f"].T,
        (tp["bih_f"] + tp["bhh_f"]).reshape(1, 4 * _TDIM),
        tp["Wih_b"].T, tp["Whh_b"].T,
        (tp["bih_b"] + tp["bhh_b"]).reshape(1, 4 * _TDIM),
        tp["ng"].reshape(1, 2 * _TDIM), tp["nb"].reshape(1, 2 * _TDIM),
        params["W1"], params["b1"].reshape(1, _HID),
        params["W2"][:_HID],
        params["n1g"].reshape(1, _HID), params["n1b"].reshape(1, _HID),
        params["W3"].reshape(1, _HID),
        params["b3"].reshape(1, 1),
    )
    return out[None, :, :_N]


# speedup vs baseline: 1.8179x; 1.8179x over previous
import functools
import math

import jax
import jax.numpy as jnp
from jax import lax
from jax.experimental import pallas as pl
from jax.experimental.pallas import tpu as pltpu
from jax.experimental.pallas import tpu_sc as plsc

_B, _S, _N, _F = 1, 8, 10000, 128
_HID, _TDIM, _FH = 128, 64, 4
_E = 160000

_BN = 512
_NPAD = 10240
_NB = _NPAD // _BN
_NSC = 16
_CHUNK = 128
_NCH = 80
_EPT = _NCH * _CHUNK
_EPAD = _NSC * _EPT
_RPT = _NPAD // _NSC


def _gelu(v):
    return 0.5 * v * (1.0 + lax.erf(v * (1.0 / math.sqrt(2.0))))


def _transform_body(apply_gelu, h_ref, w_ref, b_ref, tt_ref, s_ref):
    g = h_ref[0]
    if apply_gelu:
        g = _gelu(g)
    r0 = jnp.dot(g, w_ref[0], preferred_element_type=jnp.float32) + b_ref[0][None, :]
    r1 = jnp.dot(g, w_ref[1], preferred_element_type=jnp.float32) + b_ref[1][None, :]
    r2 = jnp.dot(g, w_ref[2], preferred_element_type=jnp.float32) + b_ref[2][None, :]
    tt_ref[0, 0] = r0
    tt_ref[0, 1] = r1
    s_ref[0] = r2


def _transform(h, w_all, b_all, apply_gelu):
    return pl.pallas_call(
        functools.partial(_transform_body, apply_gelu),
        grid=(_S, _NB),
        in_specs=[
            pl.BlockSpec((1, _BN, _HID), lambda t, n: (t, n, 0)),
            pl.BlockSpec((3, _HID, _HID), lambda t, n: (0, 0, 0)),
            pl.BlockSpec((3, _HID), lambda t, n: (0, 0)),
        ],
        out_specs=[
            pl.BlockSpec((1, 2, _BN, _HID), lambda t, n: (t, 0, n, 0)),
            pl.BlockSpec((1, _BN, _HID), lambda t, n: (t, n, 0)),
        ],
        out_shape=[
            jax.ShapeDtypeStruct((_S, 2, _NPAD, _HID), jnp.float32),
            jax.ShapeDtypeStruct((_S, _NPAD, _HID), jnp.float32),
        ],
    )(h, w_all, b_all)


def _sc_conv(tt_flat, s_all, gidx_all, dst3, w3):
    mesh = plsc.VectorSubcoreMesh(core_axis_name="c", subcore_axis_name="s")

    @functools.partial(
        pl.kernel,
        out_type=jax.ShapeDtypeStruct((_S, _NPAD, _HID), jnp.float32),
        mesh=mesh,
        scratch_types=[
            pltpu.VMEM((4, _CHUNK), jnp.int32),
            pltpu.VMEM((4, _CHUNK), jnp.int32),
            pltpu.VMEM((4, _CHUNK), jnp.float32),
            pltpu.VMEM((_CHUNK, _HID), jnp.float32),
            pltpu.VMEM((_CHUNK, _HID), jnp.float32),
            pltpu.VMEM_SHARED((_NPAD, _HID), jnp.float32),
            pltpu.SemaphoreType.DMA,
            pltpu.SemaphoreType.DMA,
            pltpu.SemaphoreType.DMA,
            pltpu.SemaphoreType.DMA,
            pltpu.SemaphoreType.DMA,
            pltpu.SemaphoreType.DMA,
        ],
    )
    def conv(tt_hbm, s_hbm, gidx_hbm, dst_hbm, w_hbm, agg_hbm,
             gi4, di4, w4, ra, rb, acc,
             gsa, gsb, ssa, ssb, isa, isb):
        cid = lax.axis_index("c")
        sid = lax.axis_index("s")
        row0 = sid * _RPT
        rows = [ra, rb]
        gsem = [gsa, gsb]
        ssem = [ssa, ssb]
        isem = [isa, isb]

        def idx_copies(t, ci, slot, copyf, sem):
            copyf(gidx_hbm.at[t, sid, ci], gi4.at[slot], sem)
            copyf(dst_hbm.at[sid, ci], di4.at[slot], sem)
            copyf(w_hbm.at[sid, ci], w4.at[slot], sem)

        def scale(buf, slot):
            def gbody(g, carry):
                w16 = w4[slot, pl.ds(g * 16, 16)]
                for j in range(16):
                    wspl = lax.gather(
                        w16, jnp.full((16, 1), j, jnp.int32),
                        lax.GatherDimensionNumbers(
                            offset_dims=(), collapsed_slice_dims=(0,),
                            start_index_map=(0,)),
                        (1,),
                        mode=lax.GatherScatterMode.PROMISE_IN_BOUNDS)
                    r = g * 16 + j
                    for fc in range(_HID // 16):
                        fsl = pl.ds(fc * 16, 16)
                        buf[r, fsl] = buf[r, fsl] * wspl
                return carry

            lax.fori_loop(0, _CHUNK // 16, gbody, 0)

        def t_body(ti, carry):
            t = cid * (_S // 2) + ti
            pltpu.sync_copy(s_hbm.at[t, pl.ds(row0, _RPT)],
                            acc.at[pl.ds(row0, _RPT)])
            plsc.subcore_barrier()

            idx_copies(t, 0, 0,
                       lambda s_, d_, m_: pltpu.sync_copy(s_, d_), None)
            pltpu.async_copy(tt_hbm.at[gi4.at[0]], rows[0], gsem[0])
            idx_copies(t, 1, 1, pltpu.async_copy, isem[1])

            def grp(g0, carry2):
                for b in range(4):
                    i = g0 * 4 + b
                    cur = b % 2
                    nxt = (b + 1) % 2
                    pltpu.make_async_copy(
                        tt_hbm.at[gi4.at[b]], rows[cur], gsem[cur]).wait()
                    scale(rows[cur], b)
                    pltpu.async_copy(rows[cur], acc.at[di4.at[b]],
                                     ssem[cur], add=True)

                    @pl.when(i >= 1)
                    def _():
                        pltpu.make_async_copy(
                            rows[nxt], acc.at[di4.at[(b + 3) % 4]],
                            ssem[nxt]).wait()

                    @pl.when(i + 1 < _NCH)
                    def _():
                        bn = (b + 1) % 4
                        pltpu.make_async_copy(
                            gidx_hbm.at[t, sid, i + 1], gi4.at[bn],
                            isem[nxt]).wait()
                        pltpu.make_async_copy(
                            dst_hbm.at[sid, i + 1], di4.at[bn],
                            isem[nxt]).wait()
                        pltpu.make_async_copy(
                            w_hbm.at[sid, i + 1], w4.at[bn],
                            isem[nxt]).wait()
                        pltpu.async_copy(
                            tt_hbm.at[gi4.at[bn]], rows[nxt], gsem[nxt])

                    @pl.when(i + 2 < _NCH)
                    def _():
                        idx_copies(t, i + 2, (b + 2) % 4,
                                   pltpu.async_copy, isem[cur])
                return carry2

            lax.fori_loop(0, _NCH // 4, grp, 0)
            pltpu.make_async_copy(
                rows[1], acc.at[di4.at[3]], ssem[1]).wait()
            plsc.subcore_barrier()
            pltpu.sync_copy(acc.at[pl.ds(row0, _RPT)],
                            agg_hbm.at[t, pl.ds(row0, _RPT)])
            plsc.subcore_barrier()
            return carry

        lax.fori_loop(0, _S // 2, t_body, 0)

    return conv(tt_flat, s_all, gidx_all, dst3, w3)


def _xw_body(x_ref, w_ref, b_ref, o_ref):
    o_ref[...] = (jnp.dot(x_ref[...], w_ref[...],
                          preferred_element_type=jnp.float32)
                  + b_ref[0][None, :])


def _xw(xlast, w2b, b2):
    return pl.pallas_call(
        _xw_body,
        grid=(_NB,),
        in_specs=[
            pl.BlockSpec((_BN, _F), lambda n: (n, 0)),
            pl.BlockSpec((_F, _HID), lambda n: (0, 0)),
            pl.BlockSpec((1, _HID), lambda n: (0, 0)),
        ],
        out_specs=pl.BlockSpec((_BN, _HID), lambda n: (n, 0)),
        out_shape=jax.ShapeDtypeStruct((_NPAD, _HID), jnp.float32),
    )(xlast, w2b, b2)


def _final_body(agg_ref, xw_ref, wip_ref, bip_ref, wihf_ref, whhf_ref, bf_ref,
                wihb_ref, whhb_ref, bb_ref, ng_ref, nbb_ref, w1_ref, b1_ref,
                w2a_ref, n1g_ref, n1b_ref, w3_ref, b3_ref, out_ref):
    zs = []
    for t in range(_S):
        g = _gelu(agg_ref[t])
        zs.append(jnp.dot(g, wip_ref[...], preferred_element_type=jnp.float32)
                  + bip_ref[0][None, :])

    def lstm(zlist, wih, whh, bsum):
        h = jnp.zeros((_BN, _TDIM), jnp.float32)
        c = jnp.zeros((_BN, _TDIM), jnp.float32)
        outs = []
        for z in zlist:
            gates = (jnp.dot(z, wih, preferred_element_type=jnp.float32)
                     + jnp.dot(h, whh, preferred_element_type=jnp.float32)
                     + bsum[0][None, :])
            ii = gates[:, :_TDIM]
            ff = gates[:, _TDIM:2 * _TDIM]
            gg = gates[:, 2 * _TDIM:3 * _TDIM]
            oo = gates[:, 3 * _TDIM:]
            c = jax.nn.sigmoid(ff) * c + jax.nn.sigmoid(ii) * jnp.tanh(gg)
            h = jax.nn.sigmoid(oo) * jnp.tanh(c)
            outs.append(h)
        return outs

    fo = lstm(zs, wihf_ref[...], whhf_ref[...], bf_ref)
    bo_rev = lstm(zs[::-1], wihb_ref[...], whhb_ref[...], bb_ref)

    def ln(v, g, b):
        m = jnp.mean(v, axis=-1, keepdims=True)
        d = v - m
        var = jnp.mean(d * d, axis=-1, keepdims=True)
        return d * lax.rsqrt(var + 1e-5) * g[0][None, :] + b[0][None, :]

    rows = []
    for fh in range(_FH):
        t = (_S - _FH) + fh
        ocat = jnp.concatenate([fo[t], bo_rev[(_S - 1) - t]], axis=1)
        oln = ln(ocat, ng_ref, nbb_ref)
        u = _gelu(jnp.dot(oln, w1_ref[...], preferred_element_type=jnp.float32)
                  + b1_ref[0][None, :])
        v = (jnp.dot(u, w2a_ref[...], preferred_element_type=jnp.float32)
             + xw_ref[:, fh, :])
        y = ln(_gelu(v), n1g_ref, n1b_ref)
        rows.append(jnp.sum(y * w3_ref[0][None, :], axis=1) + b3_ref[0, 0])
    out_ref[...] = jnp.stack(rows, axis=0)


def _final(agg, xwrep, *smalls):
    small_specs = []
    for a in smalls:
        small_specs.append(
            pl.BlockSpec(a.shape, lambda n, r=len(a.shape): (0,) * r))
    return pl.pallas_call(
        _final_body,
        grid=(_NB,),
        in_specs=[
            pl.BlockSpec((_S, _BN, _HID), lambda n: (0, n, 0)),
            pl.BlockSpec((_BN, _FH, _HID), lambda n: (n, 0, 0)),
        ] + small_specs,
        out_specs=pl.BlockSpec((_FH, _BN), lambda n: (0, n)),
        out_shape=jax.ShapeDtypeStruct((_FH, _NPAD), jnp.float32),
    )(agg, xwrep, *smalls)


def kernel(x, edge_index, edge_types, edge_weights, params):
    x = x.astype(jnp.float32)
    xp = jnp.pad(x[0], ((0, 0), (0, _NPAD - _N), (0, 0)))

    src = edge_index[0].astype(jnp.int32)
    dst = edge_index[1].astype(jnp.int32)
    ty = edge_types.astype(jnp.int32)
    pad_e = _EPAD - _E
    bidx = jnp.pad(ty * _NPAD + src, (0, pad_e))
    toffs = jnp.arange(_S, dtype=jnp.int32) * (2 * _NPAD)
    gidx_all = (bidx[None, :] + toffs[:, None]).reshape(_S, _NSC, _NCH, _CHUNK)
    dst3 = jnp.pad(dst, (0, pad_e)).reshape(_NSC, _NCH, _CHUNK)
    w3 = jnp.pad(edge_weights.astype(jnp.float32),
                 (0, pad_e)).reshape(_NSC, _NCH, _CHUNK)

    h_in = xp
    agg = None
    for li, p in enumerate(params["gnn"]):
        att = jax.nn.softmax(p["att"])
        w_all = jnp.concatenate([p["We"], p["Ws"][None]], axis=0) * att[:, None, None]
        b_all = jnp.concatenate([p["be"], p["bs"][None]], axis=0) * att[:, None]
        tt, s_all = _transform(h_in, w_all, b_all, apply_gelu=(li > 0))
        agg = _sc_conv(tt.reshape(_S * 2 * _NPAD, _HID), s_all,
                       gidx_all, dst3, w3)
        h_in = agg

    tp = params["temporal"]
    xw = _xw(xp[_S - 1], params["W2"][_HID:], params["b2"].reshape(1, _HID))
    xwv = xw[:_N]
    xwrep = jnp.concatenate(
        [xwv, xwv, xwv, xwv, xwv[: _FH * (_NPAD - _N)]], axis=0
    ).reshape(_NPAD, _FH, _HID)

    out = _final(
        agg, xwrep,
        tp["Wip"], tp["bip"].reshape(1, _TDIM),
        tp["Wih_f"].T, tp["Whh_f"].T,
        (tp["bih_f"] + tp["bhh_f"]).reshape(1, 4 * _TDIM),
        tp["Wih_b"].T, tp["Whh_b"].T,
        (tp["bih_b"] + tp["bhh_b"]).reshape(1, 4 * _TDIM),
        tp["ng"].reshape(1, 2 * _TDIM), tp["nb"].reshape(1, 2 * _TDIM),
        params["W1"], params["b1"].reshape(1, _HID),
        params["W2"][:_HID],
        params["n1g"].reshape(1, _HID), params["n1b"].reshape(1, _HID),
        params["W3"].reshape(1, _HID),
        params["b3"].reshape(1, 1),
    )
    return out[None, :, :_N]


# speedup vs baseline: 1.9383x; 1.0662x over previous
import functools
import math

import jax
import jax.numpy as jnp
from jax import lax
from jax.experimental import pallas as pl
from jax.experimental.pallas import tpu as pltpu
from jax.experimental.pallas import tpu_sc as plsc

_B, _S, _N, _F = 1, 8, 10000, 128
_HID, _TDIM, _FH = 128, 64, 4
_E = 160000

_BN = 512
_NPAD = 10240
_NB = _NPAD // _BN
_NSC = 16
_CHUNK = 128
_NCH = 80
_EPT = _NCH * _CHUNK
_EPAD = _NSC * _EPT
_RPT = _NPAD // _NSC


def _gelu(v):
    return 0.5 * v * (1.0 + lax.erf(v * (1.0 / math.sqrt(2.0))))


def _transform_body(apply_gelu, h_ref, w_ref, b_ref, tt_ref, s_ref):
    g = h_ref[0]
    if apply_gelu:
        g = _gelu(g)
    r0 = jnp.dot(g, w_ref[0], preferred_element_type=jnp.float32) + b_ref[0][None, :]
    r1 = jnp.dot(g, w_ref[1], preferred_element_type=jnp.float32) + b_ref[1][None, :]
    r2 = jnp.dot(g, w_ref[2], preferred_element_type=jnp.float32) + b_ref[2][None, :]
    tt_ref[0, 0] = r0
    tt_ref[0, 1] = r1
    s_ref[0] = r2


def _transform(h, w_all, b_all, apply_gelu):
    return pl.pallas_call(
        functools.partial(_transform_body, apply_gelu),
        grid=(_S, _NB),
        in_specs=[
            pl.BlockSpec((1, _BN, _HID), lambda t, n: (t, n, 0)),
            pl.BlockSpec((3, _HID, _HID), lambda t, n: (0, 0, 0)),
            pl.BlockSpec((3, _HID), lambda t, n: (0, 0)),
        ],
        out_specs=[
            pl.BlockSpec((1, 2, _BN, _HID), lambda t, n: (t, 0, n, 0)),
            pl.BlockSpec((1, _BN, _HID), lambda t, n: (t, n, 0)),
        ],
        out_shape=[
            jax.ShapeDtypeStruct((_S, 2, _NPAD, _HID), jnp.float32),
            jax.ShapeDtypeStruct((_S, _NPAD, _HID), jnp.float32),
        ],
    )(h, w_all, b_all)


def _sc_conv(tt_flat, s_all, pk_all, w3):
    mesh = plsc.VectorSubcoreMesh(core_axis_name="c", subcore_axis_name="s")

    @functools.partial(
        pl.kernel,
        out_type=jax.ShapeDtypeStruct((_S, _NPAD, _HID), jnp.float32),
        mesh=mesh,
        scratch_types=[
            pltpu.VMEM((4, 2, _CHUNK), jnp.int32),
            pltpu.VMEM((_NCH, _CHUNK), jnp.float32),
            pltpu.VMEM((_CHUNK, _HID), jnp.float32),
            pltpu.VMEM((_CHUNK, _HID), jnp.float32),
            pltpu.VMEM_SHARED((_NPAD, _HID), jnp.float32),
            pltpu.SemaphoreType.DMA,
            pltpu.SemaphoreType.DMA,
            pltpu.SemaphoreType.DMA,
            pltpu.SemaphoreType.DMA,
            pltpu.SemaphoreType.DMA,
            pltpu.SemaphoreType.DMA,
        ],
    )
    def conv(tt_hbm, s_hbm, pk_hbm, w_hbm, agg_hbm,
             pk4, w_v, ra, rb, acc,
             gsa, gsb, ssa, ssb, isa, isb):
        cid = lax.axis_index("c")
        sid = lax.axis_index("s")
        row0 = sid * _RPT
        rows = [ra, rb]
        gsem = [gsa, gsb]
        ssem = [ssa, ssb]
        isem = [isa, isb]

        pltpu.sync_copy(w_hbm.at[sid], w_v)

        def scale(buf, ci):
            def gbody(g, carry):
                w16 = w_v[ci, pl.ds(g * 16, 16)]
                for j in range(16):
                    wspl = lax.gather(
                        w16, jnp.full((16, 1), j, jnp.int32),
                        lax.GatherDimensionNumbers(
                            offset_dims=(), collapsed_slice_dims=(0,),
                            start_index_map=(0,)),
                        (1,),
                        mode=lax.GatherScatterMode.PROMISE_IN_BOUNDS)
                    r = g * 16 + j
                    for fc in range(_HID // 16):
                        fsl = pl.ds(fc * 16, 16)
                        buf[r, fsl] = buf[r, fsl] * wspl
                return carry

            lax.fori_loop(0, _CHUNK // 16, gbody, 0)

        def t_body(ti, carry):
            t = cid * (_S // 2) + ti
            pltpu.sync_copy(s_hbm.at[t, pl.ds(row0, _RPT)],
                            acc.at[pl.ds(row0, _RPT)])
            plsc.subcore_barrier()

            pltpu.sync_copy(pk_hbm.at[t, sid, 0], pk4.at[0])
            pltpu.async_copy(tt_hbm.at[pk4.at[0, 0]], rows[0], gsem[0])
            pltpu.async_copy(pk_hbm.at[t, sid, 1], pk4.at[1], isem[1])

            def grp(g0, carry2):
                for b in range(4):
                    i = g0 * 4 + b
                    cur = b % 2
                    nxt = (b + 1) % 2
                    pltpu.make_async_copy(
                        tt_hbm.at[pk4.at[b, 0]], rows[cur], gsem[cur]).wait()
                    scale(rows[cur], i)
                    pltpu.async_copy(rows[cur], acc.at[pk4.at[b, 1]],
                                     ssem[cur], add=True)

                    @pl.when(i >= 1)
                    def _():
                        pltpu.make_async_copy(
                            rows[nxt], acc.at[pk4.at[(b + 3) % 4, 1]],
                            ssem[nxt]).wait()

                    @pl.when(i + 1 < _NCH)
                    def _():
                        bn = (b + 1) % 4
                        pltpu.make_async_copy(
                            pk_hbm.at[t, sid, i + 1], pk4.at[bn],
                            isem[nxt]).wait()
                        pltpu.async_copy(
                            tt_hbm.at[pk4.at[bn, 0]], rows[nxt], gsem[nxt])

                    @pl.when(i + 2 < _NCH)
                    def _():
                        pltpu.async_copy(pk_hbm.at[t, sid, i + 2],
                                         pk4.at[(b + 2) % 4], isem[cur])
                return carry2

            lax.fori_loop(0, _NCH // 4, grp, 0)
            pltpu.make_async_copy(
                rows[1], acc.at[pk4.at[3, 1]], ssem[1]).wait()
            plsc.subcore_barrier()
            pltpu.sync_copy(acc.at[pl.ds(row0, _RPT)],
                            agg_hbm.at[t, pl.ds(row0, _RPT)])
            plsc.subcore_barrier()
            return carry

        lax.fori_loop(0, _S // 2, t_body, 0)

    return conv(tt_flat, s_all, pk_all, w3)


def _xw_body(x_ref, w_ref, b_ref, o_ref):
    o_ref[...] = (jnp.dot(x_ref[...], w_ref[...],
                          preferred_element_type=jnp.float32)
                  + b_ref[0][None, :])


def _xw(xlast, w2b, b2):
    return pl.pallas_call(
        _xw_body,
        grid=(_NB,),
        in_specs=[
            pl.BlockSpec((_BN, _F), lambda n: (n, 0)),
            pl.BlockSpec((_F, _HID), lambda n: (0, 0)),
            pl.BlockSpec((1, _HID), lambda n: (0, 0)),
        ],
        out_specs=pl.BlockSpec((_BN, _HID), lambda n: (n, 0)),
        out_shape=jax.ShapeDtypeStruct((_NPAD, _HID), jnp.float32),
    )(xlast, w2b, b2)


def _final_body(agg_ref, xw_ref, wip_ref, bip_ref, wihf_ref, whhf_ref, bf_ref,
                wihb_ref, whhb_ref, bb_ref, ng_ref, nbb_ref, w1_ref, b1_ref,
                w2a_ref, n1g_ref, n1b_ref, w3_ref, b3_ref, out_ref):
    zs = []
    for t in range(_S):
        g = _gelu(agg_ref[t])
        zs.append(jnp.dot(g, wip_ref[...], preferred_element_type=jnp.float32)
                  + bip_ref[0][None, :])

    def lstm(zlist, wih, whh, bsum):
        h = jnp.zeros((_BN, _TDIM), jnp.float32)
        c = jnp.zeros((_BN, _TDIM), jnp.float32)
        outs = []
        for z in zlist:
            gates = (jnp.dot(z, wih, preferred_element_type=jnp.float32)
                     + jnp.dot(h, whh, preferred_element_type=jnp.float32)
                     + bsum[0][None, :])
            ii = gates[:, :_TDIM]
            ff = gates[:, _TDIM:2 * _TDIM]
            gg = gates[:, 2 * _TDIM:3 * _TDIM]
            oo = gates[:, 3 * _TDIM:]
            c = jax.nn.sigmoid(ff) * c + jax.nn.sigmoid(ii) * jnp.tanh(gg)
            h = jax.nn.sigmoid(oo) * jnp.tanh(c)
            outs.append(h)
        return outs

    fo = lstm(zs, wihf_ref[...], whhf_ref[...], bf_ref)
    bo_rev = lstm(zs[::-1], wihb_ref[...], whhb_ref[...], bb_ref)

    def ln(v, g, b):
        m = jnp.mean(v, axis=-1, keepdims=True)
        d = v - m
        var = jnp.mean(d * d, axis=-1, keepdims=True)
        return d * lax.rsqrt(var + 1e-5) * g[0][None, :] + b[0][None, :]

    rows = []
    for fh in range(_FH):
        t = (_S - _FH) + fh
        ocat = jnp.concatenate([fo[t], bo_rev[(_S - 1) - t]], axis=1)
        oln = ln(ocat, ng_ref, nbb_ref)
        u = _gelu(jnp.dot(oln, w1_ref[...], preferred_element_type=jnp.float32)
                  + b1_ref[0][None, :])
        v = (jnp.dot(u, w2a_ref[...], preferred_element_type=jnp.float32)
             + xw_ref[:, fh, :])
        y = ln(_gelu(v), n1g_ref, n1b_ref)
        rows.append(jnp.sum(y * w3_ref[0][None, :], axis=1) + b3_ref[0, 0])
    out_ref[...] = jnp.stack(rows, axis=0)


def _final(agg, xwrep, *smalls):
    small_specs = []
    for a in smalls:
        small_specs.append(
            pl.BlockSpec(a.shape, lambda n, r=len(a.shape): (0,) * r))
    return pl.pallas_call(
        _final_body,
        grid=(_NB,),
        in_specs=[
            pl.BlockSpec((_S, _BN, _HID), lambda n: (0, n, 0)),
            pl.BlockSpec((_BN, _FH, _HID), lambda n: (n, 0, 0)),
        ] + small_specs,
        out_specs=pl.BlockSpec((_FH, _BN), lambda n: (0, n)),
        out_shape=jax.ShapeDtypeStruct((_FH, _NPAD), jnp.float32),
    )(agg, xwrep, *smalls)


def kernel(x, edge_index, edge_types, edge_weights, params):
    x = x.astype(jnp.float32)
    xp = jnp.pad(x[0], ((0, 0), (0, _NPAD - _N), (0, 0)))

    src = edge_index[0].astype(jnp.int32)
    dst = edge_index[1].astype(jnp.int32)
    ty = edge_types.astype(jnp.int32)
    pad_e = _EPAD - _E
    bidx = jnp.pad(ty * _NPAD + src, (0, pad_e))
    toffs = jnp.arange(_S, dtype=jnp.int32) * (2 * _NPAD)
    gidx_all = (bidx[None, :] + toffs[:, None]).reshape(_S, _NSC, _NCH, 1, _CHUNK)
    dst3 = jnp.pad(dst, (0, pad_e)).reshape(1, _NSC, _NCH, 1, _CHUNK)
    w3 = jnp.pad(edge_weights.astype(jnp.float32),
                 (0, pad_e)).reshape(_NSC, _NCH, _CHUNK)
    pk_all = jnp.concatenate([
        gidx_all,
        jnp.broadcast_to(dst3, (_S, _NSC, _NCH, 1, _CHUNK)),
    ], axis=3)

    h_in = xp
    agg = None
    for li, p in enumerate(params["gnn"]):
        att = jax.nn.softmax(p["att"])
        w_all = jnp.concatenate([p["We"], p["Ws"][None]], axis=0) * att[:, None, None]
        b_all = jnp.concatenate([p["be"], p["bs"][None]], axis=0) * att[:, None]
        tt, s_all = _transform(h_in, w_all, b_all, apply_gelu=(li > 0))
        agg = _sc_conv(tt.reshape(_S * 2 * _NPAD, _HID), s_all, pk_all, w3)
        h_in = agg

    tp = params["temporal"]
    xw = _xw(xp[_S - 1], params["W2"][_HID:], params["b2"].reshape(1, _HID))
    xwv = xw[:_N]
    xwrep = jnp.concatenate(
        [xwv, xwv, xwv, xwv, xwv[: _FH * (_NPAD - _N)]], axis=0
    ).reshape(_NPAD, _FH, _HID)

    out = _final(
        agg, xwrep,
        tp["Wip"], tp["bip"].reshape(1, _TDIM),
        tp["Wih_f"].T, tp["Whh_f"].T,
        (tp["bih_f"] + tp["bhh_f"]).reshape(1, 4 * _TDIM),
        tp["Wih_b"].T, tp["Whh_b"].T,
        (tp["bih_b"] + tp["bhh_b"]).reshape(1, 4 * _TDIM),
        tp["ng"].reshape(1, 2 * _TDIM), tp["nb"].reshape(1, 2 * _TDIM),
        params["W1"], params["b1"].reshape(1, _HID),
        params["W2"][:_HID],
        params["n1g"].reshape(1, _HID), params["n1b"].reshape(1, _HID),
        params["W3"].reshape(1, _HID),
        params["b3"].reshape(1, 1),
    )
    return out[None, :, :_N]
